# Initial kernel scaffold; baseline (speedup 1.0000x reference)
import jax
import jax.numpy as jnp
from jax import lax
from jax.experimental import pallas as pl
from jax.experimental.pallas import tpu as pltpu

_LINKAGE = [(15, 13), (13, 11), (16, 14), (14, 12), (11, 12), (5, 11),
            (6, 12), (5, 6), (5, 7), (6, 8), (7, 9), (8, 10), (1, 2),
            (0, 1), (0, 2), (1, 3), (2, 4), (3, 5), (4, 6)]
_PDT = 1.0
_EXPANSION = 0.3
_HW_RATIO = 2.0
_BS, _P, _J, _H, _W = 8, 10, 17, 64, 64
_L = len(_LINKAGE)
_C = 2 * _L
_TS = (_H * _W) // 128


def _body(paf_ref, j_ref, m_ref, out_ref):
    f32 = jnp.float32
    joints = j_ref[0]
    x = joints[:, :, 0]
    y = joints[:, :, 1]
    v = joints[:, :, 2]
    invis = v <= 0.0
    visp = jnp.any(v > 0.0, axis=1)

    inf = jnp.array(jnp.inf, f32)
    tlx = jnp.min(jnp.where(invis, inf, x), axis=1)
    tly = jnp.min(jnp.where(invis, inf, y), axis=1)
    brx = jnp.max(jnp.where(invis, -inf, x), axis=1)
    bry = jnp.max(jnp.where(invis, -inf, y), axis=1)
    whx = brx - tlx
    why = bry - tly
    whx = jnp.where(whx < 1.0, 1.0, whx)
    why = jnp.where(why < 1.0, 1.0, why)
    ctx = 0.5 * (brx + tlx)
    cty = 0.5 * (bry + tly)
    whx2 = jnp.maximum(whx, why / _HW_RATIO)
    why2 = jnp.maximum(why, whx / _HW_RATIO)
    ex = 0.5 + _EXPANSION
    maxx = jnp.round(ctx + ex * whx2)
    minx = jnp.round(ctx - ex * whx2)
    maxy = jnp.round(cty + ex * why2)
    miny = jnp.round(cty - ex * why2)

    def c2(a):
        return a.reshape(_P, 1)

    minx, maxx, miny, maxy = c2(minx), c2(maxx), c2(miny), c2(maxy)
    visp2 = c2(visp)

    pix = lax.broadcasted_iota(jnp.int32, (_TS, 128), 0) * 128 + \
        lax.broadcasted_iota(jnp.int32, (_TS, 128), 1)
    yf = (pix // _W).astype(f32)
    xf = (pix % _W).astype(f32)

    bits = jnp.zeros((_TS, 128), jnp.int32)
    for p in range(_P):
        inb = ((xf >= minx[p:p + 1]) & (xf <= maxx[p:p + 1]) &
               (yf >= miny[p:p + 1]) & (yf <= maxy[p:p + 1]) &
               visp2[p:p + 1])
        bits = bits + jnp.where(inb, jnp.int32(1 << p), jnp.int32(0))

    mask_t = m_ref[0]
    pbit = jnp.array([1 << p for p in range(_P)], jnp.int32).reshape(_P, 1)

    acc = jnp.zeros((), f32)
    for l, (a, b) in enumerate(_LINKAGE):
        sx = x[:, a:a + 1]
        sy = y[:, a:a + 1]
        ex_ = x[:, b:b + 1]
        ey_ = y[:, b:b + 1]
        validl = ((v[:, a:a + 1] > 0.0) & (v[:, b:b + 1] > 0.0) &
                  ((sx != ex_) | (sy != ey_)))
        vecx = jnp.where(validl, ex_ - sx, 1.0)
        vecy = jnp.where(validl, ey_ - sy, 1.0)
        norm = jnp.sqrt(vecx * vecx + vecy * vecy)
        ux = vecx / norm
        uy = vecy / norm
        sdot = sx * ux + sy * uy
        scross = sx * uy - sy * ux
        hi = jnp.where(validl, norm + _PDT, -inf)

        count = jnp.zeros((_TS, 128), f32)
        txn = jnp.zeros((_TS, 128), f32)
        tyn = jnp.zeros((_TS, 128), f32)
        for p in range(_P):
            uxp = ux[p:p + 1]
            uyp = uy[p:p + 1]
            t = xf * uxp + yf * uyp - sdot[p:p + 1]
            cr = xf * uyp - yf * uxp - scross[p:p + 1]
            m = ((t >= -_PDT) & (t <= hi[p:p + 1]) &
                 (jnp.abs(cr) <= _PDT))
            count = count + jnp.where(m, 1.0, 0.0)
            txn = txn + jnp.where(m, uxp, 0.0)
            tyn = tyn + jnp.where(m, uyp, 0.0)

        invis_l = invis[:, a:a + 1] | invis[:, b:b + 1]
        act = jnp.sum(jnp.where(invis_l & visp2, pbit, 0), axis=0,
                      keepdims=True)
        bad = (bits & act) != 0
        paf_lw = jnp.minimum(mask_t, jnp.where(bad, 0.0, 1.0))

        div = jnp.maximum(count, 1.0)
        tx = txn / div
        ty = tyn / div
        lw = jnp.where(count > 0.0, 1.0, paf_lw)
        px = paf_ref[0, 2 * l]
        py = paf_ref[0, 2 * l + 1]
        dx = px - tx
        dy = py - ty
        acc = acc + jnp.sum((dx * dx + dy * dy) * lw)

    out_ref[0] = jnp.full((128,), acc / (_C * _H * _W), f32)


def kernel(paf_pred, jointsXYV, mask):
    paf2 = paf_pred.reshape(_BS, _C, _TS, 128)
    mask2 = mask.reshape(_BS, _TS, 128)
    out = pl.pallas_call(
        _body,
        grid=(_BS,),
        in_specs=[
            pl.BlockSpec((1, _C, _TS, 128), lambda b: (b, 0, 0, 0)),
            pl.BlockSpec((1, _P, _J, 3), lambda b: (b, 0, 0)),
            pl.BlockSpec((1, _TS, 128), lambda b: (b, 0, 0)),
        ],
        out_specs=pl.BlockSpec((1, 128), lambda b: (b, 0)),
        out_shape=jax.ShapeDtypeStruct((_BS, 128), jnp.float32),
    )(paf2, jointsXYV, mask2)
    return out[:, 0]


# speedup vs baseline: 2.0492x; 2.0492x over previous
import jax
import jax.numpy as jnp
from jax import lax
from jax.experimental import pallas as pl
from jax.experimental.pallas import tpu as pltpu

_LINKAGE = [(15, 13), (13, 11), (16, 14), (14, 12), (11, 12), (5, 11),
            (6, 12), (5, 6), (5, 7), (6, 8), (7, 9), (8, 10), (1, 2),
            (0, 1), (0, 2), (1, 3), (2, 4), (3, 5), (4, 6)]
_PDT = 1.0
_EXPANSION = 0.3
_HW_RATIO = 2.0
_BS, _P, _J, _H, _W = 8, 10, 17, 64, 64
_L = len(_LINKAGE)
_C = 2 * _L
_TS = (_H * _W) // 128


def _body(paf_ref, j_ref, m_ref, out_ref):
    f32 = jnp.float32
    joints = j_ref[0]
    x = joints[:, :, 0]
    y = joints[:, :, 1]
    v = joints[:, :, 2]
    invis = v <= 0.0
    visp = jnp.any(v > 0.0, axis=1)

    inf = jnp.array(jnp.inf, f32)
    tlx = jnp.min(jnp.where(invis, inf, x), axis=1)
    tly = jnp.min(jnp.where(invis, inf, y), axis=1)
    brx = jnp.max(jnp.where(invis, -inf, x), axis=1)
    bry = jnp.max(jnp.where(invis, -inf, y), axis=1)
    whx = brx - tlx
    why = bry - tly
    whx = jnp.where(whx < 1.0, 1.0, whx)
    why = jnp.where(why < 1.0, 1.0, why)
    ctx = 0.5 * (brx + tlx)
    cty = 0.5 * (bry + tly)
    whx2 = jnp.maximum(whx, why / _HW_RATIO)
    why2 = jnp.maximum(why, whx / _HW_RATIO)
    ex = 0.5 + _EXPANSION
    maxx = jnp.round(ctx + ex * whx2)
    minx = jnp.round(ctx - ex * whx2)
    maxy = jnp.round(cty + ex * why2)
    miny = jnp.round(cty - ex * why2)

    def c2(a):
        return a.reshape(_P, 1)

    minx, maxx, miny, maxy = c2(minx), c2(maxx), c2(miny), c2(maxy)
    visp2 = c2(visp)

    pix = lax.broadcasted_iota(jnp.int32, (_TS, 128), 0) * 128 + \
        lax.broadcasted_iota(jnp.int32, (_TS, 128), 1)
    yf = (pix // _W).astype(f32)
    xf = (pix % _W).astype(f32)

    bits = jnp.zeros((_TS, 128), jnp.int32)
    for p in range(_P):
        inb = ((xf >= minx[p:p + 1]) & (xf <= maxx[p:p + 1]) &
               (yf >= miny[p:p + 1]) & (yf <= maxy[p:p + 1]) &
               visp2[p:p + 1])
        bits = bits + jnp.where(inb, jnp.int32(1 << p), jnp.int32(0))

    mask_t = m_ref[0]
    pbit = jnp.left_shift(jnp.int32(1),
                          lax.broadcasted_iota(jnp.int32, (_P, 1), 0))

    acc = jnp.zeros((), f32)
    for l, (a, b) in enumerate(_LINKAGE):
        sx = x[:, a:a + 1]
        sy = y[:, a:a + 1]
        ex_ = x[:, b:b + 1]
        ey_ = y[:, b:b + 1]
        validl = ((v[:, a:a + 1] > 0.0) & (v[:, b:b + 1] > 0.0) &
                  ((sx != ex_) | (sy != ey_)))
        vecx = jnp.where(validl, ex_ - sx, 1.0)
        vecy = jnp.where(validl, ey_ - sy, 1.0)
        norm = jnp.sqrt(vecx * vecx + vecy * vecy)
        ux = vecx / norm
        uy = vecy / norm
        sdot = sx * ux + sy * uy
        scross = sx * uy - sy * ux
        hi = jnp.where(validl, norm + _PDT, -inf)

        count = jnp.zeros((_TS, 128), f32)
        txn = jnp.zeros((_TS, 128), f32)
        tyn = jnp.zeros((_TS, 128), f32)
        for p in range(_P):
            uxp = ux[p:p + 1]
            uyp = uy[p:p + 1]
            t = xf * uxp + yf * uyp - sdot[p:p + 1]
            cr = xf * uyp - yf * uxp - scross[p:p + 1]
            m = ((t >= -_PDT) & (t <= hi[p:p + 1]) &
                 (jnp.abs(cr) <= _PDT))
            count = count + jnp.where(m, 1.0, 0.0)
            txn = txn + jnp.where(m, uxp, 0.0)
            tyn = tyn + jnp.where(m, uyp, 0.0)

        invis_l = invis[:, a:a + 1] | invis[:, b:b + 1]
        act = jnp.sum(jnp.where(invis_l & visp2, pbit, 0), axis=0,
                      keepdims=True)
        bad = (bits & act) != 0
        paf_lw = jnp.minimum(mask_t, jnp.where(bad, 0.0, 1.0))

        div = jnp.maximum(count, 1.0)
        tx = txn / div
        ty = tyn / div
        lw = jnp.where(count > 0.0, 1.0, paf_lw)
        px = paf_ref[0, 2 * l]
        py = paf_ref[0, 2 * l + 1]
        dx = px - tx
        dy = py - ty
        acc = acc + jnp.sum((dx * dx + dy * dy) * lw)

    out_ref[0, 0] = jnp.full((128,), acc / (_C * _H * _W), f32)


def kernel(paf_pred, jointsXYV, mask):
    paf2 = paf_pred.reshape(_BS, _C, _TS, 128)
    mask2 = mask.reshape(_BS, _TS, 128)
    out = pl.pallas_call(
        _body,
        grid=(_BS,),
        in_specs=[
            pl.BlockSpec((1, _C, _TS, 128), lambda b: (b, 0, 0, 0)),
            pl.BlockSpec((1, _P, _J, 3), lambda b: (b, 0, 0, 0)),
            pl.BlockSpec((1, _TS, 128), lambda b: (b, 0, 0)),
        ],
        out_specs=pl.BlockSpec((1, 1, 128), lambda b: (b, 0, 0)),
        out_shape=jax.ShapeDtypeStruct((_BS, 1, 128), jnp.float32),
    )(paf2, jointsXYV, mask2)
    return out[:, 0, 0]


# speedup vs baseline: 2.1763x; 1.0620x over previous
import functools

import jax
import jax.numpy as jnp
from jax import lax
from jax.experimental import pallas as pl
from jax.experimental.pallas import tpu as pltpu
from jax.experimental.pallas import tpu_sc as plsc

_LINKAGE = [(15, 13), (13, 11), (16, 14), (14, 12), (11, 12), (5, 11),
            (6, 12), (5, 6), (5, 7), (6, 8), (7, 9), (8, 10), (1, 2),
            (0, 1), (0, 2), (1, 3), (2, 4), (3, 5), (4, 6)]
_PDT = 1.0
_EXPANSION = 0.3
_HW_RATIO = 2.0
_BS, _P, _J, _H, _W = 8, 10, 17, 64, 64
_L = len(_LINKAGE)
_C = 2 * _L
_TS = (_H * _W) // 128
_NL = 16
_BIG = 3.0e38
_MAGIC = 12582912.0


def _round_ne(x):
    return (x + _MAGIC) - _MAGIC


def _rsqrt_newton(n2):
    i = lax.bitcast_convert_type(n2, jnp.int32)
    i = jnp.int32(0x5F3759DF) - lax.shift_right_arithmetic(i, 1)
    y = lax.bitcast_convert_type(i, jnp.float32)
    for _ in range(3):
        y = y * (1.5 - 0.5 * n2 * y * y)
    return y


def _sc_body(jt_hbm, t1_hbm, t2_hbm, jv, t1_v, t2_v):
    w = lax.axis_index("c") * 16 + lax.axis_index("s")

    @pl.when(w < _BS)
    def _():
        b = w
        pltpu.sync_copy(jt_hbm.at[b], jv)

        lane = lax.broadcasted_iota(jnp.int32, (_NL,), 0)
        one = jnp.ones((_NL,), jnp.float32)

        tlx = one * _BIG
        tly = one * _BIG
        brx = -one * _BIG
        bry = -one * _BIG
        visp = lane < 0
        for j in range(_J):
            xj = jv[j, 0]
            yj = jv[j, 1]
            vj = jv[j, 2]
            inv = vj <= 0.0
            tlx = jnp.minimum(tlx, jnp.where(inv, _BIG, xj))
            tly = jnp.minimum(tly, jnp.where(inv, _BIG, yj))
            brx = jnp.maximum(brx, jnp.where(inv, -_BIG, xj))
            bry = jnp.maximum(bry, jnp.where(inv, -_BIG, yj))
            visp = visp | (vj > 0.0)
        whx = brx - tlx
        why = bry - tly
        whx = jnp.where(whx < 1.0, 1.0, whx)
        why = jnp.where(why < 1.0, 1.0, why)
        ctx = 0.5 * (brx + tlx)
        cty = 0.5 * (bry + tly)
        whx2 = jnp.maximum(whx, why / _HW_RATIO)
        why2 = jnp.maximum(why, whx / _HW_RATIO)
        exp = jnp.float32(0.5 + _EXPANSION)
        t2_v[0] = jnp.where(visp, _round_ne(ctx - exp * whx2), _BIG)
        t2_v[1] = jnp.where(visp, _round_ne(ctx + exp * whx2), -_BIG)
        t2_v[2] = jnp.where(visp, _round_ne(cty - exp * why2), _BIG)
        t2_v[3] = jnp.where(visp, _round_ne(cty + exp * why2), -_BIG)

        for l, (a, bb) in enumerate(_LINKAGE):
            sx = jv[a, 0]
            sy = jv[a, 1]
            sv = jv[a, 2]
            ex = jv[bb, 0]
            ey = jv[bb, 1]
            ev = jv[bb, 2]
            valid = ((sv > 0.0) & (ev > 0.0) &
                     ((sx != ex) | (sy != ey)))
            vecx = jnp.where(valid, ex - sx, 1.0)
            vecy = jnp.where(valid, ey - sy, 1.0)
            n2 = vecx * vecx + vecy * vecy
            rs = _rsqrt_newton(n2)
            ux = vecx * rs
            uy = vecy * rs
            norm = n2 * rs
            t1_v[l, 0] = ux
            t1_v[l, 1] = uy
            t1_v[l, 2] = sx * ux + sy * uy
            t1_v[l, 3] = sx * uy - sy * ux
            t1_v[l, 4] = jnp.where(valid, norm + _PDT, -_BIG)
            invis_l = (sv <= 0.0) | (ev <= 0.0)
            t1_v[l, 5] = jnp.where(visp & invis_l, 1.0, 0.0)
            t1_v[l, 6] = jnp.zeros((_NL,), jnp.float32)
            t1_v[l, 7] = jnp.zeros((_NL,), jnp.float32)

        pltpu.sync_copy(t1_v, t1_hbm.at[b])
        pltpu.sync_copy(t2_v, t2_hbm.at[b])


@functools.lru_cache(maxsize=1)
def _sc_prep():
    return pl.kernel(
        _sc_body,
        out_type=[
            jax.ShapeDtypeStruct((_BS, _L, 8, _NL), jnp.float32),
            jax.ShapeDtypeStruct((_BS, 4, _NL), jnp.float32),
        ],
        mesh=plsc.VectorSubcoreMesh(core_axis_name="c",
                                    subcore_axis_name="s"),
        scratch_types=[
            pltpu.VMEM((_J, 3, _NL), jnp.float32),
            pltpu.VMEM((_L, 8, _NL), jnp.float32),
            pltpu.VMEM((4, _NL), jnp.float32),
        ],
    )


def _tc_body(paf_ref, m_ref, t1_ref, t2_ref, out_ref):
    f32 = jnp.float32
    pix = lax.broadcasted_iota(jnp.int32, (_TS, 128), 0) * 128 + \
        lax.broadcasted_iota(jnp.int32, (_TS, 128), 1)
    yf = (pix // _W).astype(f32)
    xf = (pix % _W).astype(f32)

    bits = jnp.zeros((_TS, 128), jnp.int32)
    for p in range(_P):
        inb = ((xf >= t2_ref[0, 0, p]) & (xf <= t2_ref[0, 1, p]) &
               (yf >= t2_ref[0, 2, p]) & (yf <= t2_ref[0, 3, p]))
        bits = bits + jnp.where(inb, jnp.int32(1 << p), jnp.int32(0))

    mask_t = m_ref[0]
    lacc = jnp.zeros((_TS, 128), f32)
    for l in range(_L):
        count = jnp.zeros((_TS, 128), f32)
        txn = jnp.zeros((_TS, 128), f32)
        tyn = jnp.zeros((_TS, 128), f32)
        for p in range(_P):
            hi_s = t1_ref[0, l, 4, p]

            def _yes(count=count, txn=txn, tyn=tyn, l=l, p=p, hi_s=hi_s):
                ux_s = t1_ref[0, l, 0, p]
                uy_s = t1_ref[0, l, 1, p]
                sdot_s = t1_ref[0, l, 2, p]
                scross_s = t1_ref[0, l, 3, p]
                t = xf * ux_s + yf * uy_s - sdot_s
                cr = xf * uy_s - yf * ux_s - scross_s
                m = ((t >= -_PDT) & (t <= hi_s) &
                     (jnp.abs(cr) <= _PDT))
                return (count + jnp.where(m, 1.0, 0.0),
                        txn + jnp.where(m, ux_s, 0.0),
                        tyn + jnp.where(m, uy_s, 0.0))

            def _no(count=count, txn=txn, tyn=tyn):
                return count, txn, tyn

            count, txn, tyn = lax.cond(hi_s > 0.0, _yes, _no)

        act = jnp.int32(0)
        for p in range(_P):
            act = act + jnp.where(t1_ref[0, l, 5, p] > 0.0,
                                  jnp.int32(1 << p), jnp.int32(0))
        bad = (bits & act) != 0
        paf_lw = jnp.minimum(mask_t, jnp.where(bad, 0.0, 1.0))
        div = jnp.maximum(count, 1.0)
        tx = txn / div
        ty = tyn / div
        lw = jnp.where(count > 0.0, 1.0, paf_lw)
        dx = paf_ref[0, 2 * l] - tx
        dy = paf_ref[0, 2 * l + 1] - ty
        lacc = lacc + (dx * dx + dy * dy) * lw

    out_ref[0, 0] = jnp.full((128,), jnp.sum(lacc) / (_C * _H * _W),
                             jnp.float32)


def kernel(paf_pred, jointsXYV, mask):
    jt = jnp.transpose(jointsXYV, (0, 2, 3, 1))
    jt = jnp.pad(jt, ((0, 0), (0, 0), (0, 0), (0, _NL - _P)),
                 constant_values=-1.0)
    t1, t2 = _sc_prep()(jt)

    paf2 = paf_pred.reshape(_BS, _C, _TS, 128)
    mask2 = mask.reshape(_BS, _TS, 128)
    out = pl.pallas_call(
        _tc_body,
        grid=(_BS,),
        in_specs=[
            pl.BlockSpec((1, _C, _TS, 128), lambda b: (b, 0, 0, 0)),
            pl.BlockSpec((1, _TS, 128), lambda b: (b, 0, 0)),
            pl.BlockSpec((1, _L, 8, _NL), lambda b: (b, 0, 0, 0),
                         memory_space=pltpu.SMEM),
            pl.BlockSpec((1, 4, _NL), lambda b: (b, 0, 0),
                         memory_space=pltpu.SMEM),
        ],
        out_specs=pl.BlockSpec((1, 1, 128), lambda b: (b, 0, 0)),
        out_shape=jax.ShapeDtypeStruct((_BS, 1, 128), jnp.float32),
    )(paf2, mask2, t1, t2)
    return out[:, 0, 0]


# speedup vs baseline: 2.4008x; 1.1031x over previous
import functools

import jax
import jax.numpy as jnp
from jax import lax
from jax.experimental import pallas as pl
from jax.experimental.pallas import tpu as pltpu
from jax.experimental.pallas import tpu_sc as plsc

_LINKAGE = [(15, 13), (13, 11), (16, 14), (14, 12), (11, 12), (5, 11),
            (6, 12), (5, 6), (5, 7), (6, 8), (7, 9), (8, 10), (1, 2),
            (0, 1), (0, 2), (1, 3), (2, 4), (3, 5), (4, 6)]
_PDT = 1.0
_EXPANSION = 0.3
_HW_RATIO = 2.0
_BS, _P, _J, _H, _W = 8, 10, 17, 64, 64
_L = len(_LINKAGE)
_C = 2 * _L
_TS = (_H * _W) // 128
_NL = 16
_BIG = 3.0e38
_MAGIC = 12582912.0


def _round_ne(x):
    return (x + _MAGIC) - _MAGIC


def _rsqrt_newton(n2):
    i = lax.bitcast_convert_type(n2, jnp.int32)
    i = jnp.int32(0x5F3759DF) - lax.shift_right_arithmetic(i, 1)
    y = lax.bitcast_convert_type(i, jnp.float32)
    for _ in range(3):
        y = y * (1.5 - 0.5 * n2 * y * y)
    return y


def _sc_body(jt_hbm, t1_hbm, t2_hbm, jv, t1_v, t2_v):
    w = lax.axis_index("c") * 16 + lax.axis_index("s")

    @pl.when(w < _BS)
    def _():
        b = w
        pltpu.sync_copy(jt_hbm.at[b], jv)

        lane = lax.broadcasted_iota(jnp.int32, (_NL,), 0)
        one = jnp.ones((_NL,), jnp.float32)

        tlx = one * _BIG
        tly = one * _BIG
        brx = -one * _BIG
        bry = -one * _BIG
        visp = lane < 0
        for j in range(_J):
            xj = jv[j, 0]
            yj = jv[j, 1]
            vj = jv[j, 2]
            inv = vj <= 0.0
            tlx = jnp.minimum(tlx, jnp.where(inv, _BIG, xj))
            tly = jnp.minimum(tly, jnp.where(inv, _BIG, yj))
            brx = jnp.maximum(brx, jnp.where(inv, -_BIG, xj))
            bry = jnp.maximum(bry, jnp.where(inv, -_BIG, yj))
            visp = visp | (vj > 0.0)
        whx = brx - tlx
        why = bry - tly
        whx = jnp.where(whx < 1.0, 1.0, whx)
        why = jnp.where(why < 1.0, 1.0, why)
        ctx = 0.5 * (brx + tlx)
        cty = 0.5 * (bry + tly)
        whx2 = jnp.maximum(whx, why / _HW_RATIO)
        why2 = jnp.maximum(why, whx / _HW_RATIO)
        exp = jnp.float32(0.5 + _EXPANSION)
        t2_v[0] = jnp.where(visp, _round_ne(ctx - exp * whx2), _BIG)
        t2_v[1] = jnp.where(visp, _round_ne(ctx + exp * whx2), -_BIG)
        t2_v[2] = jnp.where(visp, _round_ne(cty - exp * why2), _BIG)
        t2_v[3] = jnp.where(visp, _round_ne(cty + exp * why2), -_BIG)

        for l, (a, bb) in enumerate(_LINKAGE):
            sx = jv[a, 0]
            sy = jv[a, 1]
            sv = jv[a, 2]
            ex = jv[bb, 0]
            ey = jv[bb, 1]
            ev = jv[bb, 2]
            valid = ((sv > 0.0) & (ev > 0.0) &
                     ((sx != ex) | (sy != ey)))
            vecx = jnp.where(valid, ex - sx, 1.0)
            vecy = jnp.where(valid, ey - sy, 1.0)
            n2 = vecx * vecx + vecy * vecy
            rs = _rsqrt_newton(n2)
            ux = vecx * rs
            uy = vecy * rs
            norm = n2 * rs
            t1_v[l, 0] = ux
            t1_v[l, 1] = uy
            t1_v[l, 2] = sx * ux + sy * uy
            t1_v[l, 3] = sx * uy - sy * ux
            t1_v[l, 4] = jnp.where(valid, norm + _PDT, -_BIG)
            invis_l = (sv <= 0.0) | (ev <= 0.0)
            t1_v[l, 5] = jnp.where(visp & invis_l, 1.0, 0.0)
            t1_v[l, 6] = jnp.zeros((_NL,), jnp.float32)
            t1_v[l, 7] = jnp.zeros((_NL,), jnp.float32)

        pltpu.sync_copy(t1_v, t1_hbm.at[b])
        pltpu.sync_copy(t2_v, t2_hbm.at[b])


@functools.lru_cache(maxsize=1)
def _sc_prep():
    return pl.kernel(
        _sc_body,
        out_type=[
            jax.ShapeDtypeStruct((_BS, _L, 8, _NL), jnp.float32),
            jax.ShapeDtypeStruct((_BS, 4, _NL), jnp.float32),
        ],
        mesh=plsc.VectorSubcoreMesh(core_axis_name="c",
                                    subcore_axis_name="s"),
        scratch_types=[
            pltpu.VMEM((_J, 3, _NL), jnp.float32),
            pltpu.VMEM((_L, 8, _NL), jnp.float32),
            pltpu.VMEM((4, _NL), jnp.float32),
        ],
    )


def _tc_body(paf_ref, m_ref, t1_ref, t2_ref, out_ref):
    f32 = jnp.float32
    pix = lax.broadcasted_iota(jnp.int32, (_TS, 128), 0) * 128 + \
        lax.broadcasted_iota(jnp.int32, (_TS, 128), 1)
    yf = (pix // _W).astype(f32)
    xf = (pix % _W).astype(f32)

    bits = jnp.zeros((_TS, 128), jnp.int32)
    for p in range(_P):
        inb = ((xf >= t2_ref[0, 0, p]) & (xf <= t2_ref[0, 1, p]) &
               (yf >= t2_ref[0, 2, p]) & (yf <= t2_ref[0, 3, p]))
        bits = bits + jnp.where(inb, jnp.int32(1 << p), jnp.int32(0))

    mask_t = m_ref[0]
    lacc = jnp.zeros((_TS, 128), f32)
    for l in range(_L):
        count = jnp.zeros((_TS, 128), f32)
        txn = jnp.zeros((_TS, 128), f32)
        tyn = jnp.zeros((_TS, 128), f32)
        for p in range(_P):
            hi_s = t1_ref[0, l, 4, p]
            ux_s = t1_ref[0, l, 0, p]
            uy_s = t1_ref[0, l, 1, p]
            sdot_s = t1_ref[0, l, 2, p]
            scross_s = t1_ref[0, l, 3, p]
            t = xf * ux_s + yf * uy_s - sdot_s
            cr = xf * uy_s - yf * ux_s - scross_s
            m = ((t >= -_PDT) & (t <= hi_s) &
                 (jnp.abs(cr) <= _PDT))
            count = count + jnp.where(m, 1.0, 0.0)
            txn = txn + jnp.where(m, ux_s, 0.0)
            tyn = tyn + jnp.where(m, uy_s, 0.0)

        act = jnp.int32(0)
        for p in range(_P):
            act = act + jnp.where(t1_ref[0, l, 5, p] > 0.0,
                                  jnp.int32(1 << p), jnp.int32(0))
        bad = (bits & act) != 0
        paf_lw = jnp.minimum(mask_t, jnp.where(bad, 0.0, 1.0))
        div = jnp.maximum(count, 1.0)
        tx = txn / div
        ty = tyn / div
        lw = jnp.where(count > 0.0, 1.0, paf_lw)
        dx = paf_ref[0, 2 * l] - tx
        dy = paf_ref[0, 2 * l + 1] - ty
        lacc = lacc + (dx * dx + dy * dy) * lw

    out_ref[0, 0] = jnp.full((128,), jnp.sum(lacc) / (_C * _H * _W),
                             jnp.float32)


def kernel(paf_pred, jointsXYV, mask):
    jt = jnp.transpose(jointsXYV, (0, 2, 3, 1))
    jt = jnp.pad(jt, ((0, 0), (0, 0), (0, 0), (0, _NL - _P)),
                 constant_values=-1.0)
    t1, t2 = _sc_prep()(jt)

    paf2 = paf_pred.reshape(_BS, _C, _TS, 128)
    mask2 = mask.reshape(_BS, _TS, 128)
    out = pl.pallas_call(
        _tc_body,
        grid=(_BS,),
        in_specs=[
            pl.BlockSpec((1, _C, _TS, 128), lambda b: (b, 0, 0, 0)),
            pl.BlockSpec((1, _TS, 128), lambda b: (b, 0, 0)),
            pl.BlockSpec((1, _L, 8, _NL), lambda b: (b, 0, 0, 0),
                         memory_space=pltpu.SMEM),
            pl.BlockSpec((1, 4, _NL), lambda b: (b, 0, 0),
                         memory_space=pltpu.SMEM),
        ],
        out_specs=pl.BlockSpec((1, 1, 128), lambda b: (b, 0, 0)),
        out_shape=jax.ShapeDtypeStruct((_BS, 1, 128), jnp.float32),
    )(paf2, mask2, t1, t2)
    return out[:, 0, 0]


# speedup vs baseline: 2.4795x; 1.0328x over previous
import functools

import jax
import jax.numpy as jnp
from jax import lax
from jax.experimental import pallas as pl
from jax.experimental.pallas import tpu as pltpu
from jax.experimental.pallas import tpu_sc as plsc

_LINKAGE = [(15, 13), (13, 11), (16, 14), (14, 12), (11, 12), (5, 11),
            (6, 12), (5, 6), (5, 7), (6, 8), (7, 9), (8, 10), (1, 2),
            (0, 1), (0, 2), (1, 3), (2, 4), (3, 5), (4, 6)]
_PDT = 1.0
_EXPANSION = 0.3
_HW_RATIO = 2.0
_BS, _P, _J, _H, _W = 8, 10, 17, 64, 64
_L = len(_LINKAGE)
_C = 2 * _L
_TS = (_H * _W) // 128
_NL = 16
_TR = _L * 8 + 8
_BIG = 3.0e38
_MAGIC = 12582912.0


def _round_ne(x):
    return (x + _MAGIC) - _MAGIC


def _rsqrt_newton(n2):
    i = lax.bitcast_convert_type(n2, jnp.int32)
    i = jnp.int32(0x5F3759DF) - lax.shift_right_arithmetic(i, 1)
    y = lax.bitcast_convert_type(i, jnp.float32)
    for _ in range(3):
        y = y * (1.5 - 0.5 * n2 * y * y)
    return y


def _sc_body(jt_hbm, t_hbm, jv, t_v):
    w = lax.axis_index("c") * 16 + lax.axis_index("s")

    @pl.when(w < _BS)
    def _():
        b = w
        pltpu.sync_copy(jt_hbm.at[b], jv)

        lane = lax.broadcasted_iota(jnp.int32, (_NL,), 0)
        one = jnp.ones((_NL,), jnp.float32)

        tlx = one * _BIG
        tly = one * _BIG
        brx = -one * _BIG
        bry = -one * _BIG
        visp = lane < 0
        for j in range(_J):
            xj = jv[j, 0]
            yj = jv[j, 1]
            vj = jv[j, 2]
            inv = vj <= 0.0
            tlx = jnp.minimum(tlx, jnp.where(inv, _BIG, xj))
            tly = jnp.minimum(tly, jnp.where(inv, _BIG, yj))
            brx = jnp.maximum(brx, jnp.where(inv, -_BIG, xj))
            bry = jnp.maximum(bry, jnp.where(inv, -_BIG, yj))
            visp = visp | (vj > 0.0)
        whx = brx - tlx
        why = bry - tly
        whx = jnp.where(whx < 1.0, 1.0, whx)
        why = jnp.where(why < 1.0, 1.0, why)
        ctx = 0.5 * (brx + tlx)
        cty = 0.5 * (bry + tly)
        whx2 = jnp.maximum(whx, why / _HW_RATIO)
        why2 = jnp.maximum(why, whx / _HW_RATIO)
        exp = jnp.float32(0.5 + _EXPANSION)
        t_v[_L * 8 + 0] = jnp.where(visp, _round_ne(ctx - exp * whx2), _BIG)
        t_v[_L * 8 + 1] = jnp.where(visp, _round_ne(ctx + exp * whx2), -_BIG)
        t_v[_L * 8 + 2] = jnp.where(visp, _round_ne(cty - exp * why2), _BIG)
        t_v[_L * 8 + 3] = jnp.where(visp, _round_ne(cty + exp * why2), -_BIG)

        for l, (a, bb) in enumerate(_LINKAGE):
            sx = jv[a, 0]
            sy = jv[a, 1]
            sv = jv[a, 2]
            ex = jv[bb, 0]
            ey = jv[bb, 1]
            ev = jv[bb, 2]
            valid = ((sv > 0.0) & (ev > 0.0) &
                     ((sx != ex) | (sy != ey)))
            vecx = jnp.where(valid, ex - sx, 1.0)
            vecy = jnp.where(valid, ey - sy, 1.0)
            n2 = vecx * vecx + vecy * vecy
            rs = _rsqrt_newton(n2)
            ux = vecx * rs
            uy = vecy * rs
            norm = n2 * rs
            sdot = sx * ux + sy * uy
            scross = sx * uy - sy * ux
            r = l * 8
            t_v[r + 0] = ux
            t_v[r + 1] = uy
            t_v[r + 2] = sdot - _PDT
            t_v[r + 3] = jnp.where(valid, sdot + norm + _PDT, -_BIG)
            t_v[r + 4] = scross - _PDT
            t_v[r + 5] = scross + _PDT
            invis_l = (sv <= 0.0) | (ev <= 0.0)
            t_v[r + 6] = jnp.where(visp & invis_l, 1.0, 0.0)
            t_v[r + 7] = jnp.zeros((_NL,), jnp.float32)

        pltpu.sync_copy(t_v, t_hbm.at[b])


@functools.lru_cache(maxsize=1)
def _sc_prep():
    return pl.kernel(
        _sc_body,
        out_type=jax.ShapeDtypeStruct((_BS, _TR, _NL), jnp.float32),
        mesh=plsc.VectorSubcoreMesh(core_axis_name="c",
                                    subcore_axis_name="s"),
        scratch_types=[
            pltpu.VMEM((_J, 3, _NL), jnp.float32),
            pltpu.VMEM((_TR, _NL), jnp.float32),
        ],
    )


def _tc_body(paf_ref, m_ref, t_ref, out_ref):
    f32 = jnp.float32
    pix = lax.broadcasted_iota(jnp.int32, (_TS, 128), 0) * 128 + \
        lax.broadcasted_iota(jnp.int32, (_TS, 128), 1)
    yf = (pix // _W).astype(f32)
    xf = (pix % _W).astype(f32)

    bits = jnp.zeros((_TS, 128), jnp.int32)
    for p in range(_P):
        inb = ((xf >= t_ref[0, _L * 8 + 0, p]) &
               (xf <= t_ref[0, _L * 8 + 1, p]) &
               (yf >= t_ref[0, _L * 8 + 2, p]) &
               (yf <= t_ref[0, _L * 8 + 3, p]))
        bits = bits + jnp.where(inb, jnp.int32(1 << p), jnp.int32(0))

    mask_t = m_ref[0]
    lacc = jnp.zeros((_TS, 128), f32)
    for l in range(_L):
        r = l * 8
        count = jnp.zeros((_TS, 128), f32)
        txn = jnp.zeros((_TS, 128), f32)
        tyn = jnp.zeros((_TS, 128), f32)
        for p in range(_P):
            ux_s = t_ref[0, r + 0, p]
            uy_s = t_ref[0, r + 1, p]
            tt = xf * ux_s + yf * uy_s
            cc = xf * uy_s - yf * ux_s
            m = ((tt >= t_ref[0, r + 2, p]) & (tt <= t_ref[0, r + 3, p]) &
                 (cc >= t_ref[0, r + 4, p]) & (cc <= t_ref[0, r + 5, p]))
            count = count + jnp.where(m, 1.0, 0.0)
            txn = txn + jnp.where(m, ux_s, 0.0)
            tyn = tyn + jnp.where(m, uy_s, 0.0)

        act = jnp.int32(0)
        for p in range(_P):
            act = act + jnp.where(t_ref[0, r + 6, p] > 0.0,
                                  jnp.int32(1 << p), jnp.int32(0))
        bad = (bits & act) != 0
        paf_lw = jnp.minimum(mask_t, jnp.where(bad, 0.0, 1.0))
        div = jnp.maximum(count, 1.0)
        tx = txn / div
        ty = tyn / div
        lw = jnp.where(count > 0.0, 1.0, paf_lw)
        dx = paf_ref[0, 2 * l] - tx
        dy = paf_ref[0, 2 * l + 1] - ty
        lacc = lacc + (dx * dx + dy * dy) * lw

    out_ref[0, 0] = jnp.full((128,), jnp.sum(lacc) / (_C * _H * _W),
                             jnp.float32)


def kernel(paf_pred, jointsXYV, mask):
    jt = jnp.transpose(jointsXYV, (0, 2, 3, 1))
    jt = jnp.pad(jt, ((0, 0), (0, 0), (0, 0), (0, _NL - _P)),
                 constant_values=-1.0)
    t = _sc_prep()(jt)

    paf2 = paf_pred.reshape(_BS, _C, _TS, 128)
    mask2 = mask.reshape(_BS, _TS, 128)
    out = pl.pallas_call(
        _tc_body,
        grid=(_BS,),
        in_specs=[
            pl.BlockSpec((1, _C, _TS, 128), lambda b: (b, 0, 0, 0)),
            pl.BlockSpec((1, _TS, 128), lambda b: (b, 0, 0)),
            pl.BlockSpec((1, _TR, _NL), lambda b: (b, 0, 0),
                         memory_space=pltpu.SMEM),
        ],
        out_specs=pl.BlockSpec((1, 1, 128), lambda b: (b, 0, 0)),
        out_shape=jax.ShapeDtypeStruct((_BS, 1, 128), jnp.float32),
    )(paf2, mask2, t)
    return out[:, 0, 0]


# speedup vs baseline: 2.5485x; 1.0278x over previous
import functools

import jax
import jax.numpy as jnp
from jax import lax
from jax.experimental import pallas as pl
from jax.experimental.pallas import tpu as pltpu
from jax.experimental.pallas import tpu_sc as plsc

_LINKAGE = [(15, 13), (13, 11), (16, 14), (14, 12), (11, 12), (5, 11),
            (6, 12), (5, 6), (5, 7), (6, 8), (7, 9), (8, 10), (1, 2),
            (0, 1), (0, 2), (1, 3), (2, 4), (3, 5), (4, 6)]
_PDT = 1.0
_EXPANSION = 0.3
_HW_RATIO = 2.0
_BS, _P, _J, _H, _W = 8, 10, 17, 64, 64
_L = len(_LINKAGE)
_C = 2 * _L
_TS = (_H * _W) // 128
_NL = 16
_TR = _L * 8 + 8
_BIG = 3.0e38
_MAGIC = 12582912.0


def _round_ne(x):
    return (x + _MAGIC) - _MAGIC


def _rsqrt_newton(n2):
    i = lax.bitcast_convert_type(n2, jnp.int32)
    i = jnp.int32(0x5F3759DF) - lax.shift_right_arithmetic(i, 1)
    y = lax.bitcast_convert_type(i, jnp.float32)
    for _ in range(3):
        y = y * (1.5 - 0.5 * n2 * y * y)
    return y


def _sc_body(jt_hbm, t_hbm, jv, t_v):
    w = lax.axis_index("c") * 16 + lax.axis_index("s")

    @pl.when(w < _BS)
    def _():
        b = w
        pltpu.sync_copy(jt_hbm.at[b], jv)

        lane = lax.broadcasted_iota(jnp.int32, (_NL,), 0)
        one = jnp.ones((_NL,), jnp.float32)

        tlx = one * _BIG
        tly = one * _BIG
        brx = -one * _BIG
        bry = -one * _BIG
        visp = lane < 0
        for j in range(_J):
            xj = jv[j, 0]
            yj = jv[j, 1]
            vj = jv[j, 2]
            inv = vj <= 0.0
            tlx = jnp.minimum(tlx, jnp.where(inv, _BIG, xj))
            tly = jnp.minimum(tly, jnp.where(inv, _BIG, yj))
            brx = jnp.maximum(brx, jnp.where(inv, -_BIG, xj))
            bry = jnp.maximum(bry, jnp.where(inv, -_BIG, yj))
            visp = visp | (vj > 0.0)
        whx = brx - tlx
        why = bry - tly
        whx = jnp.where(whx < 1.0, 1.0, whx)
        why = jnp.where(why < 1.0, 1.0, why)
        ctx = 0.5 * (brx + tlx)
        cty = 0.5 * (bry + tly)
        whx2 = jnp.maximum(whx, why / _HW_RATIO)
        why2 = jnp.maximum(why, whx / _HW_RATIO)
        exp = jnp.float32(0.5 + _EXPANSION)
        t_v[_L * 8 + 0] = jnp.where(visp, _round_ne(ctx - exp * whx2), _BIG)
        t_v[_L * 8 + 1] = jnp.where(visp, _round_ne(ctx + exp * whx2), -_BIG)
        t_v[_L * 8 + 2] = jnp.where(visp, _round_ne(cty - exp * why2), _BIG)
        t_v[_L * 8 + 3] = jnp.where(visp, _round_ne(cty + exp * why2), -_BIG)

        for l, (a, bb) in enumerate(_LINKAGE):
            sx = jv[a, 0]
            sy = jv[a, 1]
            sv = jv[a, 2]
            ex = jv[bb, 0]
            ey = jv[bb, 1]
            ev = jv[bb, 2]
            valid = ((sv > 0.0) & (ev > 0.0) &
                     ((sx != ex) | (sy != ey)))
            vecx = jnp.where(valid, ex - sx, 1.0)
            vecy = jnp.where(valid, ey - sy, 1.0)
            n2 = vecx * vecx + vecy * vecy
            rs = _rsqrt_newton(n2)
            ux = vecx * rs
            uy = vecy * rs
            norm = n2 * rs
            sdot = sx * ux + sy * uy
            scross = sx * uy - sy * ux
            r = l * 8
            t_v[r + 0] = ux
            t_v[r + 1] = uy
            t_v[r + 2] = sdot - _PDT
            t_v[r + 3] = jnp.where(valid, sdot + norm + _PDT, -_BIG)
            t_v[r + 4] = scross - _PDT
            t_v[r + 5] = scross + _PDT
            invis_l = (sv <= 0.0) | (ev <= 0.0)
            t_v[r + 6] = jnp.where(visp & invis_l, 1.0, 0.0)
            t_v[r + 7] = jnp.zeros((_NL,), jnp.float32)

        pltpu.sync_copy(t_v, t_hbm.at[b])


@functools.lru_cache(maxsize=1)
def _sc_prep():
    return pl.kernel(
        _sc_body,
        out_type=jax.ShapeDtypeStruct((_BS, _TR, _NL), jnp.float32),
        mesh=plsc.VectorSubcoreMesh(core_axis_name="c",
                                    subcore_axis_name="s",
                                    num_cores=1),
        scratch_types=[
            pltpu.VMEM((_J, 3, _NL), jnp.float32),
            pltpu.VMEM((_TR, _NL), jnp.float32),
        ],
    )


def _tc_body(paf_ref, m_ref, t_ref, out_ref):
    f32 = jnp.float32
    pix = lax.broadcasted_iota(jnp.int32, (_TS, 128), 0) * 128 + \
        lax.broadcasted_iota(jnp.int32, (_TS, 128), 1)
    yf = (pix // _W).astype(f32)
    xf = (pix % _W).astype(f32)

    bits = jnp.zeros((_TS, 128), jnp.int32)
    for p in range(_P):
        inb = ((xf >= t_ref[0, _L * 8 + 0, p]) &
               (xf <= t_ref[0, _L * 8 + 1, p]) &
               (yf >= t_ref[0, _L * 8 + 2, p]) &
               (yf <= t_ref[0, _L * 8 + 3, p]))
        bits = bits + jnp.where(inb, jnp.int32(1 << p), jnp.int32(0))

    mask_t = m_ref[0]
    lacc = jnp.zeros((_TS, 128), f32)
    for l in range(_L):
        r = l * 8
        count = jnp.zeros((_TS, 128), f32)
        txn = jnp.zeros((_TS, 128), f32)
        tyn = jnp.zeros((_TS, 128), f32)
        for p in range(_P):
            ux_s = t_ref[0, r + 0, p]
            uy_s = t_ref[0, r + 1, p]
            tt = xf * ux_s + yf * uy_s
            cc = xf * uy_s - yf * ux_s
            m = ((tt >= t_ref[0, r + 2, p]) & (tt <= t_ref[0, r + 3, p]) &
                 (cc >= t_ref[0, r + 4, p]) & (cc <= t_ref[0, r + 5, p]))
            count = count + jnp.where(m, 1.0, 0.0)
            txn = txn + jnp.where(m, ux_s, 0.0)
            tyn = tyn + jnp.where(m, uy_s, 0.0)

        act = jnp.int32(0)
        for p in range(_P):
            act = act + jnp.where(t_ref[0, r + 6, p] > 0.0,
                                  jnp.int32(1 << p), jnp.int32(0))
        bad = (bits & act) != 0
        paf_lw = jnp.minimum(mask_t, jnp.where(bad, 0.0, 1.0))
        div = jnp.maximum(count, 1.0)
        tx = txn / div
        ty = tyn / div
        lw = jnp.where(count > 0.0, 1.0, paf_lw)
        dx = paf_ref[0, 2 * l] - tx
        dy = paf_ref[0, 2 * l + 1] - ty
        lacc = lacc + (dx * dx + dy * dy) * lw

    out_ref[0, 0] = jnp.full((128,), jnp.sum(lacc) / (_C * _H * _W),
                             jnp.float32)


def kernel(paf_pred, jointsXYV, mask):
    jt = jnp.transpose(jointsXYV, (0, 2, 3, 1))
    jt = jnp.pad(jt, ((0, 0), (0, 0), (0, 0), (0, _NL - _P)),
                 constant_values=-1.0)
    t = _sc_prep()(jt)

    paf2 = paf_pred.reshape(_BS, _C, _TS, 128)
    mask2 = mask.reshape(_BS, _TS, 128)
    out = pl.pallas_call(
        _tc_body,
        grid=(_BS,),
        in_specs=[
            pl.BlockSpec((1, _C, _TS, 128), lambda b: (b, 0, 0, 0)),
            pl.BlockSpec((1, _TS, 128), lambda b: (b, 0, 0)),
            pl.BlockSpec((1, _TR, _NL), lambda b: (b, 0, 0),
                         memory_space=pltpu.SMEM),
        ],
        out_specs=pl.BlockSpec((1, 1, 128), lambda b: (b, 0, 0)),
        out_shape=jax.ShapeDtypeStruct((_BS, 1, 128), jnp.float32),
    )(paf2, mask2, t)
    return out[:, 0, 0]


# speedup vs baseline: 2.5634x; 1.0058x over previous
import functools

import jax
import jax.numpy as jnp
from jax import lax
from jax.experimental import pallas as pl
from jax.experimental.pallas import tpu as pltpu
from jax.experimental.pallas import tpu_sc as plsc

_LINKAGE = [(15, 13), (13, 11), (16, 14), (14, 12), (11, 12), (5, 11),
            (6, 12), (5, 6), (5, 7), (6, 8), (7, 9), (8, 10), (1, 2),
            (0, 1), (0, 2), (1, 3), (2, 4), (3, 5), (4, 6)]
_PDT = 1.0
_EXPANSION = 0.3
_HW_RATIO = 2.0
_BS, _P, _J, _H, _W = 8, 10, 17, 64, 64
_L = len(_LINKAGE)
_C = 2 * _L
_TS = (_H * _W) // 128
_NL = 16
_TR = _L * 7 + 4
_NP = 10
_BIG = 3.0e38
_MAGIC = 12582912.0


def _round_ne(x):
    return (x + _MAGIC) - _MAGIC


def _rsqrt_newton(n2):
    i = lax.bitcast_convert_type(n2, jnp.int32)
    i = jnp.int32(0x5F3759DF) - lax.shift_right_arithmetic(i, 1)
    y = lax.bitcast_convert_type(i, jnp.float32)
    for _ in range(3):
        y = y * (1.5 - 0.5 * n2 * y * y)
    return y


def _sc_body(jt_hbm, t_hbm, jv, t_v):
    w = lax.axis_index("c") * 16 + lax.axis_index("s")

    @pl.when(w < _BS)
    def _():
        b = w
        pltpu.sync_copy(jt_hbm.at[b], jv)

        lane = lax.broadcasted_iota(jnp.int32, (_NL,), 0)
        one = jnp.ones((_NL,), jnp.float32)

        tlx = one * _BIG
        tly = one * _BIG
        brx = -one * _BIG
        bry = -one * _BIG
        visp = lane < 0
        for j in range(_J):
            xj = jv[j, 0]
            yj = jv[j, 1]
            vj = jv[j, 2]
            inv = vj <= 0.0
            tlx = jnp.minimum(tlx, jnp.where(inv, _BIG, xj))
            tly = jnp.minimum(tly, jnp.where(inv, _BIG, yj))
            brx = jnp.maximum(brx, jnp.where(inv, -_BIG, xj))
            bry = jnp.maximum(bry, jnp.where(inv, -_BIG, yj))
            visp = visp | (vj > 0.0)
        whx = brx - tlx
        why = bry - tly
        whx = jnp.where(whx < 1.0, 1.0, whx)
        why = jnp.where(why < 1.0, 1.0, why)
        ctx = 0.5 * (brx + tlx)
        cty = 0.5 * (bry + tly)
        whx2 = jnp.maximum(whx, why / _HW_RATIO)
        why2 = jnp.maximum(why, whx / _HW_RATIO)
        exp = jnp.float32(0.5 + _EXPANSION)
        t_v[_L * 7 + 0] = jnp.where(visp, _round_ne(ctx - exp * whx2), _BIG)
        t_v[_L * 7 + 1] = jnp.where(visp, _round_ne(ctx + exp * whx2), -_BIG)
        t_v[_L * 7 + 2] = jnp.where(visp, _round_ne(cty - exp * why2), _BIG)
        t_v[_L * 7 + 3] = jnp.where(visp, _round_ne(cty + exp * why2), -_BIG)

        for l, (a, bb) in enumerate(_LINKAGE):
            sx = jv[a, 0]
            sy = jv[a, 1]
            sv = jv[a, 2]
            ex = jv[bb, 0]
            ey = jv[bb, 1]
            ev = jv[bb, 2]
            valid = ((sv > 0.0) & (ev > 0.0) &
                     ((sx != ex) | (sy != ey)))
            vecx = jnp.where(valid, ex - sx, 1.0)
            vecy = jnp.where(valid, ey - sy, 1.0)
            n2 = vecx * vecx + vecy * vecy
            rs = _rsqrt_newton(n2)
            ux = vecx * rs
            uy = vecy * rs
            norm = n2 * rs
            sdot = sx * ux + sy * uy
            scross = sx * uy - sy * ux
            r = l * 7
            t_v[r + 0] = ux
            t_v[r + 1] = uy
            t_v[r + 2] = sdot - _PDT
            t_v[r + 3] = jnp.where(valid, sdot + norm + _PDT, -_BIG)
            t_v[r + 4] = scross - _PDT
            t_v[r + 5] = scross + _PDT
            invis_l = (sv <= 0.0) | (ev <= 0.0)
            t_v[r + 6] = jnp.where(visp & invis_l, 1.0, 0.0)

        pltpu.sync_copy(t_v, t_hbm.at[b])


@functools.lru_cache(maxsize=1)
def _sc_prep():
    return pl.kernel(
        _sc_body,
        out_type=jax.ShapeDtypeStruct((_BS, _TR, _NL), jnp.float32),
        mesh=plsc.VectorSubcoreMesh(core_axis_name="c",
                                    subcore_axis_name="s",
                                    num_cores=1),
        scratch_types=[
            pltpu.VMEM((_J, 3, _NL), jnp.float32),
            pltpu.VMEM((_TR, _NL), jnp.float32),
        ],
    )


def _tc_body(paf_ref, m_ref, t_ref, out_ref):
    f32 = jnp.float32
    pix = lax.broadcasted_iota(jnp.int32, (_TS, 128), 0) * 128 + \
        lax.broadcasted_iota(jnp.int32, (_TS, 128), 1)
    yf = (pix // _W).astype(f32)
    xf = (pix % _W).astype(f32)

    bits = jnp.zeros((_TS, 128), jnp.int32)
    for p in range(_P):
        inb = ((xf >= t_ref[0, _L * 7 + 0, p]) &
               (xf <= t_ref[0, _L * 7 + 1, p]) &
               (yf >= t_ref[0, _L * 7 + 2, p]) &
               (yf <= t_ref[0, _L * 7 + 3, p]))
        bits = bits + jnp.where(inb, jnp.int32(1 << p), jnp.int32(0))

    mask_t = m_ref[0]
    lacc = jnp.zeros((_TS, 128), f32)
    for l in range(_L):
        r = l * 7
        count = jnp.zeros((_TS, 128), f32)
        txn = jnp.zeros((_TS, 128), f32)
        tyn = jnp.zeros((_TS, 128), f32)
        for p in range(_P):
            ux_s = t_ref[0, r + 0, p]
            uy_s = t_ref[0, r + 1, p]
            tt = xf * ux_s + yf * uy_s
            cc = xf * uy_s - yf * ux_s
            m = ((tt >= t_ref[0, r + 2, p]) & (tt <= t_ref[0, r + 3, p]) &
                 (cc >= t_ref[0, r + 4, p]) & (cc <= t_ref[0, r + 5, p]))
            count = count + jnp.where(m, 1.0, 0.0)
            txn = txn + jnp.where(m, ux_s, 0.0)
            tyn = tyn + jnp.where(m, uy_s, 0.0)

        act = jnp.int32(0)
        for p in range(_P):
            act = act + jnp.where(t_ref[0, r + 6, p] > 0.0,
                                  jnp.int32(1 << p), jnp.int32(0))
        bad = (bits & act) != 0
        paf_lw = jnp.minimum(mask_t, jnp.where(bad, 0.0, 1.0))
        div = jnp.maximum(count, 1.0)
        tx = txn / div
        ty = tyn / div
        lw = jnp.where(count > 0.0, 1.0, paf_lw)
        dx = paf_ref[0, 2 * l] - tx
        dy = paf_ref[0, 2 * l + 1] - ty
        lacc = lacc + (dx * dx + dy * dy) * lw

    out_ref[0, 0] = jnp.full((128,), jnp.sum(lacc) / (_C * _H * _W),
                             jnp.float32)


def kernel(paf_pred, jointsXYV, mask):
    jt = jnp.transpose(jointsXYV, (0, 2, 3, 1))
    jt = jnp.pad(jt, ((0, 0), (0, 0), (0, 0), (0, _NL - _P)),
                 constant_values=-1.0)
    t = _sc_prep()(jt)

    paf2 = paf_pred.reshape(_BS, _C, _TS, 128)
    mask2 = mask.reshape(_BS, _TS, 128)
    out = pl.pallas_call(
        _tc_body,
        grid=(_BS,),
        in_specs=[
            pl.BlockSpec((1, _C, _TS, 128), lambda b: (b, 0, 0, 0)),
            pl.BlockSpec((1, _TS, 128), lambda b: (b, 0, 0)),
            pl.BlockSpec((1, _TR, _NL), lambda b: (b, 0, 0),
                         memory_space=pltpu.SMEM),
        ],
        out_specs=pl.BlockSpec((1, 1, 128), lambda b: (b, 0, 0)),
        out_shape=jax.ShapeDtypeStruct((_BS, 1, 128), jnp.float32),
    )(paf2, mask2, t)
    return out[:, 0, 0]
